# Initial kernel scaffold; baseline (speedup 1.0000x reference)
import jax
import jax.numpy as jnp
from jax.experimental import pallas as pl


def kernel(grid, vol0, vol1, vol2):
    raise NotImplementedError("write your pallas kernel here")


# speedup vs baseline: 2.8822x; 2.8822x over previous
import functools

import jax
import jax.numpy as jnp
from jax import lax
from jax.experimental import pallas as pl
from jax.experimental.pallas import tpu as pltpu
from jax.experimental.pallas import tpu_sc as plsc

P = 524288
NC = 2
NS = 16
NW = NC * NS
PPW = P // NW
CHUNK = 128
NCHUNKS = PPW // CHUNK
LANES = 16
NG = CHUNK // LANES

VOLS = ((64, 8, 1, 0, 0), (128, 4, 2, 1, 8), (256, 2, 4, 2, 12))
CTOT = 14
RPL = 64


def _sc_body(t0, t1, t2, gx, gy, gz, out, *refs):
    cx, cy, cz, w0, w1, w2 = refs[:6]
    irefs = (refs[6:14], refs[14:22], refs[22:30])
    orefs = (refs[30:32], refs[32:34], refs[34:36])
    brefs = (refs[36:44], refs[44:52], refs[52:60])
    ob, sem = refs[60], refs[61]
    wid = lax.axis_index("s") * NC + lax.axis_index("c")
    wbase = wid * PPW
    tabs = (t0, t1, t2)
    wrefs = (w0, w1, w2)
    lane = lax.iota(jnp.int32, LANES)

    def chunk_body(ci, carry):
        pbase = wbase + ci * CHUNK
        pltpu.sync_copy(gx.at[pl.ds(pbase, CHUNK)], cx)
        pltpu.sync_copy(gy.at[pl.ds(pbase, CHUNK)], cy)
        pltpu.sync_copy(gz.at[pl.ds(pbase, CHUNK)], cz)

        def build(g, c2):
            s = g * LANES
            gxv = cx[pl.ds(s, LANES)]
            gyv = cy[pl.ds(s, LANES)]
            gzv = cz[pl.ds(s, LANES)]
            for v, (n, c, grp, sh, _cb) in enumerate(VOLS):
                scale = jnp.float32(0.5 * (n - 1))
                hi = jnp.float32(n - 1)
                tx = jnp.minimum(jnp.maximum((gxv + 1.0) * scale, 0.0), hi)
                ty = jnp.minimum(jnp.maximum((gyv + 1.0) * scale, 0.0), hi)
                tz = jnp.minimum(jnp.maximum((gzv + 1.0) * scale, 0.0), hi)
                x0 = tx.astype(jnp.int32)
                y0 = ty.astype(jnp.int32)
                z0 = tz.astype(jnp.int32)
                wrefs[v][0, pl.ds(s, LANES)] = tx - x0.astype(jnp.float32)
                wrefs[v][1, pl.ds(s, LANES)] = ty - y0.astype(jnp.float32)
                wrefs[v][2, pl.ds(s, LANES)] = tz - z0.astype(jnp.float32)
                x1 = jnp.minimum(x0 + 1, n - 1)
                y1 = jnp.minimum(y0 + 1, n - 1)
                z1 = jnp.minimum(z0 + 1, n - 1)
                xa = lax.shift_right_logical(x0, sh)
                xb = lax.shift_right_logical(x1, sh)
                orefs[v][0][pl.ds(s, LANES)] = (x0 & (grp - 1)) * c
                orefs[v][1][pl.ds(s, LANES)] = (x1 & (grp - 1)) * c
                l00 = (z0 * n + y0) * RPL
                l01 = (z0 * n + y1) * RPL
                l10 = (z1 * n + y0) * RPL
                l11 = (z1 * n + y1) * RPL
                iref = irefs[v]
                iref[0][pl.ds(s, LANES)] = l00 + xa
                iref[1][pl.ds(s, LANES)] = l00 + xb
                iref[2][pl.ds(s, LANES)] = l01 + xa
                iref[3][pl.ds(s, LANES)] = l01 + xb
                iref[4][pl.ds(s, LANES)] = l10 + xa
                iref[5][pl.ds(s, LANES)] = l10 + xb
                iref[6][pl.ds(s, LANES)] = l11 + xa
                iref[7][pl.ds(s, LANES)] = l11 + xb
            return c2

        lax.fori_loop(0, NG, build, 0)

        handles = []
        for v in range(3):
            for j in range(8):
                handles.append(
                    pltpu.async_copy(tabs[v].at[irefs[v][j]],
                                     brefs[v][j], sem))
        for h in handles:
            h.wait()

        def interp(g, c2):
            s = g * LANES
            pv = lane + s
            for v, (n, c, grp, sh, cb) in enumerate(VOLS):
                wx = wrefs[v][0, pl.ds(s, LANES)]
                wy = wrefs[v][1, pl.ds(s, LANES)]
                wz = wrefs[v][2, pl.ds(s, LANES)]
                oa = orefs[v][0][pl.ds(s, LANES)]
                obv = orefs[v][1][pl.ds(s, LANES)]
                bufs = brefs[v]
                for ch in range(c):
                    cav = oa + ch
                    cbv = obv + ch

                    def gat(j, cv):
                        return plsc.load_gather(bufs[j], [pv, cv])

                    c000 = gat(0, cav)
                    c001 = gat(1, cbv)
                    c010 = gat(2, cav)
                    c011 = gat(3, cbv)
                    c100 = gat(4, cav)
                    c101 = gat(5, cbv)
                    c110 = gat(6, cav)
                    c111 = gat(7, cbv)
                    c00 = c000 + wx * (c001 - c000)
                    c01 = c010 + wx * (c011 - c010)
                    c10 = c100 + wx * (c101 - c100)
                    c11 = c110 + wx * (c111 - c110)
                    c0 = c00 + wy * (c01 - c00)
                    c1 = c10 + wy * (c11 - c10)
                    ob[cb + ch, pl.ds(s, LANES)] = c0 + wz * (c1 - c0)
            return c2

        lax.fori_loop(0, NG, interp, 0)

        pltpu.sync_copy(ob, out.at[:, pl.ds(pbase, CHUNK)])
        return carry

    lax.fori_loop(0, NCHUNKS, chunk_body, 0)


@jax.jit
def kernel(grid, vol0, vol1, vol2):
    g = grid.reshape(P, 3)
    gx, gy, gz = g[:, 0], g[:, 1], g[:, 2]
    t0 = jnp.transpose(vol0[0], (1, 2, 3, 0)).reshape(-1, 8)
    t1 = jnp.transpose(vol1[0], (1, 2, 3, 0)).reshape(-1, 8)
    t2 = jnp.transpose(vol2[0], (1, 2, 3, 0)).reshape(-1, 8)

    mesh = plsc.VectorSubcoreMesh(core_axis_name="c", subcore_axis_name="s")
    run = functools.partial(
        pl.kernel,
        mesh=mesh,
        out_type=jax.ShapeDtypeStruct((CTOT, P), jnp.float32),
        scratch_types=[
            pltpu.VMEM((CHUNK,), jnp.float32),
            pltpu.VMEM((CHUNK,), jnp.float32),
            pltpu.VMEM((CHUNK,), jnp.float32),
            pltpu.VMEM((3, CHUNK), jnp.float32),
            pltpu.VMEM((3, CHUNK), jnp.float32),
            pltpu.VMEM((3, CHUNK), jnp.float32),
            *[pltpu.VMEM((CHUNK,), jnp.int32) for _ in range(24)],
            *[pltpu.VMEM((CHUNK,), jnp.int32) for _ in range(6)],
            *[pltpu.VMEM((CHUNK, 8), jnp.float32) for _ in range(24)],
            pltpu.VMEM((CTOT, CHUNK), jnp.float32),
            pltpu.SemaphoreType.DMA,
        ],
        compiler_params=pltpu.CompilerParams(
            needs_layout_passes=False, use_tc_tiling_on_sc=False),
    )(_sc_body)
    out = run(t0, t1, t2, gx, gy, gz)
    return out.reshape(1, CTOT, 1, 1, P)


# speedup vs baseline: 16.8454x; 5.8447x over previous
import functools

import jax
import jax.numpy as jnp
from jax import lax
from jax.experimental import pallas as pl
from jax.experimental.pallas import tpu as pltpu
from jax.experimental.pallas import tpu_sc as plsc

P = 524288
NC = 2
NS = 16
NW = NC * NS
PPW = P // NW
CHUNK = 128
NCHUNKS = PPW // CHUNK
LANES = 16
NG = CHUNK // LANES

VOLS = ((64, 8, 1, 0, 0), (128, 4, 2, 1, 8))
CTOT = 14
RPL = 64
N2 = 256
V2ROWS = N2 * N2 * (N2 // 8)


def _sc_body(t0, t1, t2, gx, gy, gz, out, *refs):
    cx, cy, cz, w0, w1, w2 = refs[:6]
    irefs = (refs[6:14], refs[14:22])
    i2 = refs[22:38]
    orefs = (refs[38:40], refs[40:42])
    oa2, ob2 = refs[42], refs[43]
    brefs = (refs[44:52], refs[52:60])
    b2 = refs[60:76]
    ob, sem = refs[76], refs[77]
    wid = lax.axis_index("s") * NC + lax.axis_index("c")
    wbase = wid * PPW
    tabs = (t0, t1)
    wrefs = (w0, w1)
    lane = lax.iota(jnp.int32, LANES)

    def chunk_body(ci, carry):
        pbase = wbase + ci * CHUNK
        pltpu.sync_copy(gx.at[pl.ds(pbase, CHUNK)], cx)
        pltpu.sync_copy(gy.at[pl.ds(pbase, CHUNK)], cy)
        pltpu.sync_copy(gz.at[pl.ds(pbase, CHUNK)], cz)

        def build(g, c2):
            s = g * LANES
            gxv = cx[pl.ds(s, LANES)]
            gyv = cy[pl.ds(s, LANES)]
            gzv = cz[pl.ds(s, LANES)]
            for v, (n, c, grp, sh, _cb) in enumerate(VOLS):
                scale = jnp.float32(0.5 * (n - 1))
                hi = jnp.float32(n - 1)
                tx = jnp.minimum(jnp.maximum((gxv + 1.0) * scale, 0.0), hi)
                ty = jnp.minimum(jnp.maximum((gyv + 1.0) * scale, 0.0), hi)
                tz = jnp.minimum(jnp.maximum((gzv + 1.0) * scale, 0.0), hi)
                x0 = tx.astype(jnp.int32)
                y0 = ty.astype(jnp.int32)
                z0 = tz.astype(jnp.int32)
                wrefs[v][0, pl.ds(s, LANES)] = tx - x0.astype(jnp.float32)
                wrefs[v][1, pl.ds(s, LANES)] = ty - y0.astype(jnp.float32)
                wrefs[v][2, pl.ds(s, LANES)] = tz - z0.astype(jnp.float32)
                x1 = jnp.minimum(x0 + 1, n - 1)
                y1 = jnp.minimum(y0 + 1, n - 1)
                z1 = jnp.minimum(z0 + 1, n - 1)
                xa = lax.shift_right_logical(x0, sh)
                xb = lax.shift_right_logical(x1, sh)
                orefs[v][0][pl.ds(s, LANES)] = (x0 & (grp - 1)) * c
                orefs[v][1][pl.ds(s, LANES)] = (x1 & (grp - 1)) * c
                l00 = (z0 * n + y0) * RPL
                l01 = (z0 * n + y1) * RPL
                l10 = (z1 * n + y0) * RPL
                l11 = (z1 * n + y1) * RPL
                iref = irefs[v]
                iref[0][pl.ds(s, LANES)] = l00 + xa
                iref[1][pl.ds(s, LANES)] = l00 + xb
                iref[2][pl.ds(s, LANES)] = l01 + xa
                iref[3][pl.ds(s, LANES)] = l01 + xb
                iref[4][pl.ds(s, LANES)] = l10 + xa
                iref[5][pl.ds(s, LANES)] = l10 + xb
                iref[6][pl.ds(s, LANES)] = l11 + xa
                iref[7][pl.ds(s, LANES)] = l11 + xb
            scale = jnp.float32(0.5 * (N2 - 1))
            hi = jnp.float32(N2 - 1)
            tx = jnp.minimum(jnp.maximum((gxv + 1.0) * scale, 0.0), hi)
            ty = jnp.minimum(jnp.maximum((gyv + 1.0) * scale, 0.0), hi)
            tz = jnp.minimum(jnp.maximum((gzv + 1.0) * scale, 0.0), hi)
            x0 = tx.astype(jnp.int32)
            y0 = ty.astype(jnp.int32)
            z0 = tz.astype(jnp.int32)
            w2[0, pl.ds(s, LANES)] = tx - x0.astype(jnp.float32)
            w2[1, pl.ds(s, LANES)] = ty - y0.astype(jnp.float32)
            w2[2, pl.ds(s, LANES)] = tz - z0.astype(jnp.float32)
            x1 = jnp.minimum(x0 + 1, N2 - 1)
            y1 = jnp.minimum(y0 + 1, N2 - 1)
            z1 = jnp.minimum(z0 + 1, N2 - 1)
            oa2[pl.ds(s, LANES)] = x0 & 7
            ob2[pl.ds(s, LANES)] = x1 & 7
            xa = lax.shift_right_logical(x0, 3)
            xb = lax.shift_right_logical(x1, 3)
            l00 = (z0 * N2 + y0) * (N2 // 8)
            l01 = (z0 * N2 + y1) * (N2 // 8)
            l10 = (z1 * N2 + y0) * (N2 // 8)
            l11 = (z1 * N2 + y1) * (N2 // 8)
            rows = (l00 + xa, l00 + xb, l01 + xa, l01 + xb,
                    l10 + xa, l10 + xb, l11 + xa, l11 + xb)
            for j, r in enumerate(rows):
                i2[j][pl.ds(s, LANES)] = r
                i2[8 + j][pl.ds(s, LANES)] = r + V2ROWS
            return c2

        lax.fori_loop(0, NG, build, 0)

        handles = []
        for v in range(2):
            for j in range(8):
                handles.append(
                    pltpu.async_copy(tabs[v].at[irefs[v][j]],
                                     brefs[v][j], sem))
        for j in range(16):
            handles.append(pltpu.async_copy(t2.at[i2[j]], b2[j], sem))
        for h in handles:
            h.wait()

        def interp(g, c2):
            s = g * LANES
            pv = lane + s

            def lerp3(vals, wx, wy, wz):
                c000, c001, c010, c011, c100, c101, c110, c111 = vals
                c00 = c000 + wx * (c001 - c000)
                c01 = c010 + wx * (c011 - c010)
                c10 = c100 + wx * (c101 - c100)
                c11 = c110 + wx * (c111 - c110)
                c0 = c00 + wy * (c01 - c00)
                c1 = c10 + wy * (c11 - c10)
                return c0 + wz * (c1 - c0)

            for v, (n, c, grp, sh, cb) in enumerate(VOLS):
                wx = wrefs[v][0, pl.ds(s, LANES)]
                wy = wrefs[v][1, pl.ds(s, LANES)]
                wz = wrefs[v][2, pl.ds(s, LANES)]
                oa = orefs[v][0][pl.ds(s, LANES)]
                obv = orefs[v][1][pl.ds(s, LANES)]
                bufs = brefs[v]
                for ch in range(c):
                    cav = oa + ch
                    cbv = obv + ch

                    vals = tuple(
                        plsc.load_gather(bufs[j], [pv, cav if j % 2 == 0 else cbv])
                        for j in range(8))
                    ob[cb + ch, pl.ds(s, LANES)] = lerp3(vals, wx, wy, wz)
            wx = w2[0, pl.ds(s, LANES)]
            wy = w2[1, pl.ds(s, LANES)]
            wz = w2[2, pl.ds(s, LANES)]
            oav = oa2[pl.ds(s, LANES)]
            obv = ob2[pl.ds(s, LANES)]
            for ch in range(2):
                vals = tuple(
                    plsc.load_gather(b2[ch * 8 + j],
                                     [pv, oav if j % 2 == 0 else obv])
                    for j in range(8))
                ob[12 + ch, pl.ds(s, LANES)] = lerp3(vals, wx, wy, wz)
            return c2

        lax.fori_loop(0, NG, interp, 0)

        pltpu.sync_copy(ob, out.at[:, pl.ds(pbase, CHUNK)])
        return carry

    lax.fori_loop(0, NCHUNKS, chunk_body, 0)


@jax.jit
def kernel(grid, vol0, vol1, vol2):
    g = grid.reshape(P, 3)
    gx, gy, gz = g[:, 0], g[:, 1], g[:, 2]
    t0 = jnp.transpose(vol0[0], (1, 2, 3, 0)).reshape(-1, 8)
    t1 = jnp.transpose(vol1[0], (1, 2, 3, 0)).reshape(-1, 8)
    t2 = vol2.reshape(-1, 8)

    mesh = plsc.VectorSubcoreMesh(core_axis_name="c", subcore_axis_name="s")
    run = functools.partial(
        pl.kernel,
        mesh=mesh,
        out_type=jax.ShapeDtypeStruct((CTOT, P), jnp.float32),
        scratch_types=[
            pltpu.VMEM((CHUNK,), jnp.float32),
            pltpu.VMEM((CHUNK,), jnp.float32),
            pltpu.VMEM((CHUNK,), jnp.float32),
            pltpu.VMEM((3, CHUNK), jnp.float32),
            pltpu.VMEM((3, CHUNK), jnp.float32),
            pltpu.VMEM((3, CHUNK), jnp.float32),
            *[pltpu.VMEM((CHUNK,), jnp.int32) for _ in range(32)],
            *[pltpu.VMEM((CHUNK,), jnp.int32) for _ in range(6)],
            *[pltpu.VMEM((CHUNK, 8), jnp.float32) for _ in range(32)],
            pltpu.VMEM((CTOT, CHUNK), jnp.float32),
            pltpu.SemaphoreType.DMA,
        ],
        compiler_params=pltpu.CompilerParams(
            needs_layout_passes=False, use_tc_tiling_on_sc=False),
    )(_sc_body)
    out = run(t0, t1, t2, gx, gy, gz)
    return out.reshape(1, CTOT, 1, 1, P)


# speedup vs baseline: 43.8371x; 2.6023x over previous
import functools

import jax
import jax.numpy as jnp
from jax import lax
from jax.experimental import pallas as pl
from jax.experimental.pallas import tpu as pltpu
from jax.experimental.pallas import tpu_sc as plsc

P = 524288
NC = 2
NS = 16
NW = NC * NS
PPW = P // NW
CHUNK = 128
NCHUNKS = PPW // CHUNK
LANES = 16
NG = CHUNK // LANES

VOLS = ((64, 8, 1, 0, 0), (128, 4, 2, 1, 8))
CTOT = 14
RPL = 64
N2 = 256
V2ROWS = N2 * N2 * (N2 // 8)


V0 = 64 * 64 * 64
V1 = 128 * 128 * 128
K0 = 1024
K1 = 2048


def _fmt_body(v0, v1, t0f, t1f, inb0, inb1, outb, sem):
    wid = lax.axis_index("s") * NC + lax.axis_index("c")
    lane = lax.iota(jnp.int32, LANES)

    def do_vol(vin, tout, inb, c, k, nchunks, wvox):
        cvec = lane & (c - 1)
        vbase = (lane >> {8: 3, 4: 2}[c]) * 1
        step = 16 // c

        def chunk(ci, carry):
            base = wid * wvox + ci * k
            handles = [pltpu.async_copy(vin.at[ch, pl.ds(base, k)],
                                        inb.at[ch], sem) for ch in range(c)]
            for h in handles:
                h.wait()

            def vloop(m, c2):
                for u in range(8):
                    mm = m * 8 + u
                    vox = vbase + mm * step
                    outb[pl.ds(mm * LANES, LANES)] = plsc.load_gather(
                        inb, [cvec, vox])
                return c2

            lax.fori_loop(0, k * c // (LANES * 8), vloop, 0)
            pltpu.sync_copy(outb.at[pl.ds(0, k * c)],
                            tout.at[pl.ds(base * c, k * c)])
            return carry

        lax.fori_loop(0, nchunks, chunk, 0)

    do_vol(v0, t0f, inb0, 8, K0, V0 // NW // K0, V0 // NW)
    do_vol(v1, t1f, inb1, 4, K1, V1 // NW // K1, V1 // NW)


def _sc_body(t0, t1, t2, gx, gy, gz, out, *refs):
    cx, cy, cz, w0, w1, w2 = refs[:6]
    irefs = (refs[6:14], refs[14:22])
    i2 = refs[22:38]
    orefs = (refs[38:40], refs[40:42])
    oa2, ob2 = refs[42], refs[43]
    brefs = (refs[44:52], refs[52:60])
    b2 = refs[60:76]
    ob, sem = refs[76], refs[77]
    wid = lax.axis_index("s") * NC + lax.axis_index("c")
    wbase = wid * PPW
    tabs = (t0, t1)
    wrefs = (w0, w1)
    lane = lax.iota(jnp.int32, LANES)

    def chunk_body(ci, carry):
        pbase = wbase + ci * CHUNK
        pltpu.sync_copy(gx.at[pl.ds(pbase, CHUNK)], cx)
        pltpu.sync_copy(gy.at[pl.ds(pbase, CHUNK)], cy)
        pltpu.sync_copy(gz.at[pl.ds(pbase, CHUNK)], cz)

        def build(g, c2):
            s = g * LANES
            gxv = cx[pl.ds(s, LANES)]
            gyv = cy[pl.ds(s, LANES)]
            gzv = cz[pl.ds(s, LANES)]
            for v, (n, c, grp, sh, _cb) in enumerate(VOLS):
                scale = jnp.float32(0.5 * (n - 1))
                hi = jnp.float32(n - 1)
                tx = jnp.minimum(jnp.maximum((gxv + 1.0) * scale, 0.0), hi)
                ty = jnp.minimum(jnp.maximum((gyv + 1.0) * scale, 0.0), hi)
                tz = jnp.minimum(jnp.maximum((gzv + 1.0) * scale, 0.0), hi)
                x0 = tx.astype(jnp.int32)
                y0 = ty.astype(jnp.int32)
                z0 = tz.astype(jnp.int32)
                wrefs[v][0, pl.ds(s, LANES)] = tx - x0.astype(jnp.float32)
                wrefs[v][1, pl.ds(s, LANES)] = ty - y0.astype(jnp.float32)
                wrefs[v][2, pl.ds(s, LANES)] = tz - z0.astype(jnp.float32)
                x1 = jnp.minimum(x0 + 1, n - 1)
                y1 = jnp.minimum(y0 + 1, n - 1)
                z1 = jnp.minimum(z0 + 1, n - 1)
                xa = lax.shift_right_logical(x0, sh)
                xb = lax.shift_right_logical(x1, sh)
                orefs[v][0][pl.ds(s, LANES)] = (x0 & (grp - 1)) * c
                orefs[v][1][pl.ds(s, LANES)] = (x1 & (grp - 1)) * c
                l00 = (z0 * n + y0) * RPL
                l01 = (z0 * n + y1) * RPL
                l10 = (z1 * n + y0) * RPL
                l11 = (z1 * n + y1) * RPL
                iref = irefs[v]
                iref[0][pl.ds(s, LANES)] = l00 + xa
                iref[1][pl.ds(s, LANES)] = l00 + xb
                iref[2][pl.ds(s, LANES)] = l01 + xa
                iref[3][pl.ds(s, LANES)] = l01 + xb
                iref[4][pl.ds(s, LANES)] = l10 + xa
                iref[5][pl.ds(s, LANES)] = l10 + xb
                iref[6][pl.ds(s, LANES)] = l11 + xa
                iref[7][pl.ds(s, LANES)] = l11 + xb
            scale = jnp.float32(0.5 * (N2 - 1))
            hi = jnp.float32(N2 - 1)
            tx = jnp.minimum(jnp.maximum((gxv + 1.0) * scale, 0.0), hi)
            ty = jnp.minimum(jnp.maximum((gyv + 1.0) * scale, 0.0), hi)
            tz = jnp.minimum(jnp.maximum((gzv + 1.0) * scale, 0.0), hi)
            x0 = tx.astype(jnp.int32)
            y0 = ty.astype(jnp.int32)
            z0 = tz.astype(jnp.int32)
            w2[0, pl.ds(s, LANES)] = tx - x0.astype(jnp.float32)
            w2[1, pl.ds(s, LANES)] = ty - y0.astype(jnp.float32)
            w2[2, pl.ds(s, LANES)] = tz - z0.astype(jnp.float32)
            x1 = jnp.minimum(x0 + 1, N2 - 1)
            y1 = jnp.minimum(y0 + 1, N2 - 1)
            z1 = jnp.minimum(z0 + 1, N2 - 1)
            oa2[pl.ds(s, LANES)] = x0 & 7
            ob2[pl.ds(s, LANES)] = x1 & 7
            xa = lax.shift_right_logical(x0, 3)
            xb = lax.shift_right_logical(x1, 3)
            l00 = (z0 * N2 + y0) * (N2 // 8)
            l01 = (z0 * N2 + y1) * (N2 // 8)
            l10 = (z1 * N2 + y0) * (N2 // 8)
            l11 = (z1 * N2 + y1) * (N2 // 8)
            rows = (l00 + xa, l00 + xb, l01 + xa, l01 + xb,
                    l10 + xa, l10 + xb, l11 + xa, l11 + xb)
            for j, r in enumerate(rows):
                i2[j][pl.ds(s, LANES)] = r
                i2[8 + j][pl.ds(s, LANES)] = r + V2ROWS
            return c2

        lax.fori_loop(0, NG, build, 0)

        handles = []
        for v in range(2):
            for j in range(8):
                handles.append(
                    pltpu.async_copy(tabs[v].at[irefs[v][j]],
                                     brefs[v][j], sem))
        for j in range(16):
            handles.append(pltpu.async_copy(t2.at[i2[j]], b2[j], sem))
        for h in handles:
            h.wait()

        def interp(g, c2):
            s = g * LANES
            pv = lane + s

            def lerp3(vals, wx, wy, wz):
                c000, c001, c010, c011, c100, c101, c110, c111 = vals
                c00 = c000 + wx * (c001 - c000)
                c01 = c010 + wx * (c011 - c010)
                c10 = c100 + wx * (c101 - c100)
                c11 = c110 + wx * (c111 - c110)
                c0 = c00 + wy * (c01 - c00)
                c1 = c10 + wy * (c11 - c10)
                return c0 + wz * (c1 - c0)

            for v, (n, c, grp, sh, cb) in enumerate(VOLS):
                wx = wrefs[v][0, pl.ds(s, LANES)]
                wy = wrefs[v][1, pl.ds(s, LANES)]
                wz = wrefs[v][2, pl.ds(s, LANES)]
                oa = orefs[v][0][pl.ds(s, LANES)]
                obv = orefs[v][1][pl.ds(s, LANES)]
                bufs = brefs[v]
                for ch in range(c):
                    cav = oa + ch
                    cbv = obv + ch

                    vals = tuple(
                        plsc.load_gather(bufs[j], [pv, cav if j % 2 == 0 else cbv])
                        for j in range(8))
                    ob[cb + ch, pl.ds(s, LANES)] = lerp3(vals, wx, wy, wz)
            wx = w2[0, pl.ds(s, LANES)]
            wy = w2[1, pl.ds(s, LANES)]
            wz = w2[2, pl.ds(s, LANES)]
            oav = oa2[pl.ds(s, LANES)]
            obv = ob2[pl.ds(s, LANES)]
            for ch in range(2):
                vals = tuple(
                    plsc.load_gather(b2[ch * 8 + j],
                                     [pv, oav if j % 2 == 0 else obv])
                    for j in range(8))
                ob[12 + ch, pl.ds(s, LANES)] = lerp3(vals, wx, wy, wz)
            return c2

        lax.fori_loop(0, NG, interp, 0)

        pltpu.sync_copy(ob, out.at[:, pl.ds(pbase, CHUNK)])
        return carry

    lax.fori_loop(0, NCHUNKS, chunk_body, 0)


@jax.jit
def kernel(grid, vol0, vol1, vol2):
    g = grid.reshape(P, 3)
    gx, gy, gz = g[:, 0], g[:, 1], g[:, 2]
    t2 = vol2.reshape(-1, 8)

    mesh = plsc.VectorSubcoreMesh(core_axis_name="c", subcore_axis_name="s")
    fmt = functools.partial(
        pl.kernel,
        mesh=mesh,
        out_type=(jax.ShapeDtypeStruct((V0 * 8,), jnp.float32),
                  jax.ShapeDtypeStruct((V1 * 4,), jnp.float32)),
        scratch_types=[
            pltpu.VMEM((8, K0), jnp.float32),
            pltpu.VMEM((4, K1), jnp.float32),
            pltpu.VMEM((8192,), jnp.float32),
            pltpu.SemaphoreType.DMA,
        ],
        compiler_params=pltpu.CompilerParams(
            needs_layout_passes=False, use_tc_tiling_on_sc=False),
    )(_fmt_body)
    t0f, t1f = fmt(vol0.reshape(8, V0), vol1.reshape(4, V1))
    t0 = t0f.reshape(-1, 8)
    t1 = t1f.reshape(-1, 8)
    run = functools.partial(
        pl.kernel,
        mesh=mesh,
        out_type=jax.ShapeDtypeStruct((CTOT, P), jnp.float32),
        scratch_types=[
            pltpu.VMEM((CHUNK,), jnp.float32),
            pltpu.VMEM((CHUNK,), jnp.float32),
            pltpu.VMEM((CHUNK,), jnp.float32),
            pltpu.VMEM((3, CHUNK), jnp.float32),
            pltpu.VMEM((3, CHUNK), jnp.float32),
            pltpu.VMEM((3, CHUNK), jnp.float32),
            *[pltpu.VMEM((CHUNK,), jnp.int32) for _ in range(32)],
            *[pltpu.VMEM((CHUNK,), jnp.int32) for _ in range(6)],
            *[pltpu.VMEM((CHUNK, 8), jnp.float32) for _ in range(32)],
            pltpu.VMEM((CTOT, CHUNK), jnp.float32),
            pltpu.SemaphoreType.DMA,
        ],
        compiler_params=pltpu.CompilerParams(
            needs_layout_passes=False, use_tc_tiling_on_sc=False),
    )(_sc_body)
    out = run(t0, t1, t2, gx, gy, gz)
    return out.reshape(1, CTOT, 1, 1, P)
